# baseline (speedup 1.0000x reference)
import functools

import jax
import jax.numpy as jnp
import numpy as np
from jax.experimental import pallas as pl
from jax.experimental.pallas import tpu as pltpu

S = 32
B = 8
SL = S - 1
HID = 512
EMB = 256
VOCAB = 5000
VPAD = 5120
VBLK = 512
NPAIR = 480
NREAL = 465
INIT_P = 465
MROWS = NPAIR * B
MBLK = 960
NEG = -1e30

_pi = np.array([i for i in range(SL) for j in range(i + 1, S) if j < SL],
               dtype=np.int32)
_pj = np.array([j for i in range(SL) for j in range(i + 1, S) if j < SL],
               dtype=np.int32)
_ii = np.concatenate([_pi, np.zeros(NPAIR - NREAL, np.int32)])
_jj = np.concatenate([_pj, np.zeros(NPAIR - NREAL, np.int32)])
_tgt_rows = np.concatenate([_pj + 1, np.ones(NPAIR - NREAL, np.int32)])

_OHI = np.zeros((NPAIR, S), np.float32)
_OHI[np.arange(NPAIR), _ii] = 1.0
_OHJ = np.zeros((NPAIR, S), np.float32)
_OHJ[np.arange(NPAIR), _jj] = 1.0
_SCT = np.zeros((S * S, NPAIR), np.float32)
_SCT[_pj * S + _pi, np.arange(NREAL)] = 1.0


def _embed_body(sent_ref, emb_ref, out_ref):
    v = pl.program_id(0)

    @pl.when(v == 0)
    def _():
        out_ref[...] = jnp.zeros_like(out_ref)

    cols = jax.lax.broadcasted_iota(jnp.int32, (S * B, 1000), 1) + v * 1000
    onehot = (cols == sent_ref[...]).astype(jnp.float32)
    out_ref[...] += jnp.dot(onehot, emb_ref[...],
                            preferred_element_type=jnp.float32)


def _prep_body(e_ref, we_ref, be_ref, wt1_ref, ww1_ref,
               at_ref, bt_ref, aw_ref, bw_ref):
    h = jnp.tanh(jnp.dot(e_ref[...], we_ref[...],
                         preferred_element_type=jnp.float32) + be_ref[...])
    at_ref[...] = jnp.dot(h, wt1_ref[0:HID, :],
                          preferred_element_type=jnp.float32)
    bt_ref[...] = jnp.dot(h, wt1_ref[HID:2 * HID, :],
                          preferred_element_type=jnp.float32)
    aw_ref[...] = jnp.dot(h, ww1_ref[0:HID, :],
                          preferred_element_type=jnp.float32)
    bw_ref[...] = jnp.dot(h, ww1_ref[HID:2 * HID, :],
                          preferred_element_type=jnp.float32)


def _tgrid_body(at_ref, bt_ref, af_ref, bf_ref, bt1_ref, wt2_ref, bt2_ref,
                out_ref, outT_ref):
    b1 = bt1_ref[...][None, :, :]
    bf3 = bf_ref[...].reshape(S, B, HID)
    x = jnp.tanh(bf3 + at_ref[...][None, :, :] + b1)
    t = jnp.dot(x.reshape(S * B, HID), wt2_ref[...],
                preferred_element_type=jnp.float32) + bt2_ref[0, 0]
    out_ref[...] = t.reshape(1, S, B)
    af3 = af_ref[...].reshape(S, B, HID)
    xT = jnp.tanh(af3 + bt_ref[...][None, :, :] + b1)
    tT = jnp.dot(xT.reshape(S * B, HID), wt2_ref[...],
                 preferred_element_type=jnp.float32) + bt2_ref[0, 0]
    outT_ref[...] = tT.reshape(1, S, B)


def _hidpre_body(ohi_ref, ohj_ref, aw_ref, bw_ref, bw1_ref, out_ref):
    ps = (jnp.dot(ohi_ref[...], aw_ref[...],
                  preferred_element_type=jnp.float32) +
          jnp.dot(ohj_ref[...], bw_ref[...],
                  preferred_element_type=jnp.float32))
    out_ref[...] = jnp.tanh(ps + bw1_ref[...]).astype(jnp.bfloat16)


def _word_body(hid_ref, w2_ref, b2_ref, tgt_ref, se_ref, tl_ref, bs_ref):
    v = pl.program_id(1)

    @pl.when(v == 0)
    def _():
        se_ref[...] = jnp.zeros_like(se_ref)
        tl_ref[...] = jnp.zeros_like(tl_ref)
        bs_ref[...] = jnp.zeros_like(bs_ref)

    logits = jnp.dot(hid_ref[...], w2_ref[...],
                     preferred_element_type=jnp.float32)
    b2col = b2_ref[0]
    se_ref[0] += jnp.dot(jnp.exp(logits), jnp.exp(b2col),
                         preferred_element_type=jnp.float32)
    cols = jax.lax.broadcasted_iota(jnp.int32, (MBLK, VBLK), 1) + v * VBLK
    tmask = cols == tgt_ref[0]
    ones = jnp.ones((VBLK, 1), jnp.float32)
    tl_ref[0] += jnp.dot(jnp.where(tmask, logits, 0.0), ones,
                         preferred_element_type=jnp.float32)
    bs_ref[0] += jnp.dot(tmask.astype(jnp.float32), b2col,
                         preferred_element_type=jnp.float32)


def _scatter_body(se_ref, tl_ref, bs_ref, sct_ref, wp_ref, t01_ref):
    wp_all = tl_ref[...] + bs_ref[...] - jnp.log(se_ref[...])
    wp_ref[...] = jnp.dot(sct_ref[...], wp_all,
                          preferred_element_type=jnp.float32)
    t01_ref[...] = wp_all[INIT_P:INIT_P + 1, :]


def _dp_body(tg_ref, tgT_ref, wpT_ref, t01_ref, out_ref):
    wpT = wpT_ref[...]
    tg2 = tg_ref[...]
    tgT2 = tgT_ref[...]
    shT = -jnp.log(1.0 + jnp.exp(tgT2))
    rrN = -jnp.log(1.0 + jnp.exp(-tg2))

    krow = jax.lax.broadcasted_iota(jnp.int32, (S, S * B), 0)
    icol = jax.lax.broadcasted_iota(jnp.int32, (S, S * B), 1) // B
    validT = (krow > icol) & (krow <= SL - 1)
    wplT = jnp.where(validT, shT + wpT, NEG)

    t01c = t01_ref[...]
    negf = jnp.full((S, S * B), NEG)
    tk = jnp.where((krow - icol == 1) & (icol >= 1), 0.0, negf)
    tk = jnp.where((krow == 1) & (icol == 0), t01c, tk)
    tn = jnp.where((icol - krow == 1) & (krow >= 1), 0.0, negf)
    tn = jnp.where((krow == 0) & (icol == 1), t01c, tn)

    def body(g, carry):
        tk, tn = carry
        nts = pltpu.roll(tn + rrN, -B * g, 1)
        masked = jnp.where((krow > icol) & (krow < icol + g),
                           tk + wplT + nts, NEG)
        m = jnp.max(masked, axis=0, keepdims=True)
        s = jnp.sum(jnp.exp(masked - m), axis=0, keepdims=True)
        new = m + jnp.log(s)
        tk = jnp.where(krow - icol == g, new, tk)
        tn = jnp.where(icol - krow == g, pltpu.roll(new, B * g, 1), tn)
        return tk, tn

    _, tn = jax.lax.fori_loop(2, SL + 1, body, (tk, tn))
    out_ref[...] = tn[0:1, SL * B:S * B] + rrN[0:1, SL * B:S * B]


def kernel(sentence, emb_table, W_e, b_e, W_t1, b_t1, W_t2, b_t2,
           W_w1, b_w1, W_w2, b_w2):
    f32 = jnp.float32
    sent_flat = sentence.reshape(S * B, 1)

    e = pl.pallas_call(
        _embed_body,
        grid=(5,),
        in_specs=[
            pl.BlockSpec((S * B, 1), lambda v: (0, 0)),
            pl.BlockSpec((1000, EMB), lambda v: (v, 0)),
        ],
        out_specs=pl.BlockSpec((S * B, EMB), lambda v: (0, 0)),
        out_shape=jax.ShapeDtypeStruct((S * B, EMB), f32),
        compiler_params=pltpu.CompilerParams(
            dimension_semantics=("arbitrary",)),
    )(sent_flat, emb_table)

    at, bt, aw, bw = pl.pallas_call(
        _prep_body,
        out_shape=[jax.ShapeDtypeStruct((S * B, HID), f32)] * 4,
    )(e, W_e, b_e.reshape(1, HID), W_t1, W_w1)

    tgrid, tgridT = pl.pallas_call(
        _tgrid_body,
        grid=(S,),
        in_specs=[
            pl.BlockSpec((B, HID), lambda i: (i, 0)),
            pl.BlockSpec((B, HID), lambda i: (i, 0)),
            pl.BlockSpec((S * B, HID), lambda i: (0, 0)),
            pl.BlockSpec((S * B, HID), lambda i: (0, 0)),
            pl.BlockSpec((1, HID), lambda i: (0, 0)),
            pl.BlockSpec((HID, 1), lambda i: (0, 0)),
            pl.BlockSpec((1, 1), lambda i: (0, 0)),
        ],
        out_specs=[
            pl.BlockSpec((1, S, B), lambda i: (i, 0, 0)),
            pl.BlockSpec((1, S, B), lambda i: (i, 0, 0)),
        ],
        out_shape=[jax.ShapeDtypeStruct((S, S, B), f32)] * 2,
    )(at, bt, at, bt, b_t1.reshape(1, HID), W_t2, b_t2.reshape(1, 1))

    hidpre4 = pl.pallas_call(
        _hidpre_body,
        grid=(4,),
        in_specs=[
            pl.BlockSpec((NPAIR // 4, S), lambda p: (p, 0)),
            pl.BlockSpec((NPAIR // 4, S), lambda p: (p, 0)),
            pl.BlockSpec((S, B * HID), lambda p: (0, 0)),
            pl.BlockSpec((S, B * HID), lambda p: (0, 0)),
            pl.BlockSpec((1, B * HID), lambda p: (0, 0)),
        ],
        out_specs=pl.BlockSpec((NPAIR // 4, B * HID), lambda p: (p, 0)),
        out_shape=jax.ShapeDtypeStruct((NPAIR, B * HID), jnp.bfloat16),
    )(jnp.asarray(_OHI), jnp.asarray(_OHJ),
      aw.reshape(S, B * HID), bw.reshape(S, B * HID),
      jnp.tile(b_w1, B).reshape(1, B * HID))

    hidpre = hidpre4.reshape(MROWS, HID)
    w2p = jnp.pad(W_w2.astype(jnp.bfloat16), ((0, 0), (0, VPAD - VOCAB)))
    b2p = jnp.concatenate([b_w2, jnp.full((VPAD - VOCAB,), -1e9, f32)])
    b2p3 = b2p.reshape(VPAD // VBLK, VBLK, 1)
    tgt = sentence[jnp.asarray(_tgt_rows)]
    tgt3 = tgt.reshape(MROWS // MBLK, MBLK, 1)

    sumexp, tgtl, bsel = pl.pallas_call(
        _word_body,
        grid=(MROWS // MBLK, VPAD // VBLK),
        in_specs=[
            pl.BlockSpec((MBLK, HID), lambda m, v: (m, 0)),
            pl.BlockSpec((HID, VBLK), lambda m, v: (0, v)),
            pl.BlockSpec((1, VBLK, 1), lambda m, v: (v, 0, 0)),
            pl.BlockSpec((1, MBLK, 1), lambda m, v: (m, 0, 0)),
        ],
        out_specs=[
            pl.BlockSpec((1, MBLK, 1), lambda m, v: (m, 0, 0)),
            pl.BlockSpec((1, MBLK, 1), lambda m, v: (m, 0, 0)),
            pl.BlockSpec((1, MBLK, 1), lambda m, v: (m, 0, 0)),
        ],
        out_shape=[jax.ShapeDtypeStruct((MROWS // MBLK, MBLK, 1), f32)] * 3,
        compiler_params=pltpu.CompilerParams(
            dimension_semantics=("parallel", "arbitrary")),
    )(hidpre, w2p, b2p3, tgt3)

    wpg, t01 = pl.pallas_call(
        _scatter_body,
        out_shape=[jax.ShapeDtypeStruct((S * S, B), f32),
                   jax.ShapeDtypeStruct((1, B), f32)],
    )(sumexp.reshape(NPAIR, B), tgtl.reshape(NPAIR, B),
      bsel.reshape(NPAIR, B), jnp.asarray(_SCT))

    out = pl.pallas_call(
        _dp_body,
        out_shape=jax.ShapeDtypeStruct((1, B), f32),
    )(tgrid.reshape(S, S * B), tgridT.reshape(S, S * B),
      wpg.reshape(S, S * B), jnp.tile(t01, (1, S)))
    return out.reshape(B)


# speedup vs baseline: 1.1700x; 1.1700x over previous
import functools

import jax
import jax.numpy as jnp
import numpy as np
from jax.experimental import pallas as pl
from jax.experimental.pallas import tpu as pltpu

S = 32
B = 8
SL = S - 1
HID = 512
EMB = 256
VOCAB = 5000
VPAD = 5120
VBLK = 512
NPAIR = 480
NREAL = 465
INIT_P = 465
MROWS = NPAIR * B
MBLK = 960
NEG = -1e30

_pi = np.array([i for i in range(SL) for j in range(i + 1, S) if j < SL],
               dtype=np.int32)
_pj = np.array([j for i in range(SL) for j in range(i + 1, S) if j < SL],
               dtype=np.int32)
_ii = np.concatenate([_pi, np.zeros(NPAIR - NREAL, np.int32)])
_jj = np.concatenate([_pj, np.zeros(NPAIR - NREAL, np.int32)])
_tgt_rows = np.concatenate([_pj + 1, np.ones(NPAIR - NREAL, np.int32)])

_OHI = np.zeros((NPAIR, S), np.float32)
_OHI[np.arange(NPAIR), _ii] = 1.0
_OHJ = np.zeros((NPAIR, S), np.float32)
_OHJ[np.arange(NPAIR), _jj] = 1.0
_SCT = np.zeros((S * S, NPAIR), np.float32)
_SCT[_pj * S + _pi, np.arange(NREAL)] = 1.0


def _embed_body(sent_ref, emb_ref, out_ref):
    v = pl.program_id(0)

    @pl.when(v == 0)
    def _():
        out_ref[...] = jnp.zeros_like(out_ref)

    cols = jax.lax.broadcasted_iota(jnp.int32, (S * B, 1000), 1) + v * 1000
    onehot = (cols == sent_ref[...]).astype(jnp.float32)
    out_ref[...] += jnp.dot(onehot, emb_ref[...],
                            preferred_element_type=jnp.float32)


def _prep_body(e_ref, we_ref, be_ref, wt1_ref, ww1_ref,
               at_ref, bt_ref, aw_ref, bw_ref):
    h = jnp.tanh(jnp.dot(e_ref[...], we_ref[...],
                         preferred_element_type=jnp.float32) + be_ref[...])
    at_ref[...] = jnp.dot(h, wt1_ref[0:HID, :],
                          preferred_element_type=jnp.float32)
    bt_ref[...] = jnp.dot(h, wt1_ref[HID:2 * HID, :],
                          preferred_element_type=jnp.float32)
    aw_ref[...] = jnp.dot(h, ww1_ref[0:HID, :],
                          preferred_element_type=jnp.float32)
    bw_ref[...] = jnp.dot(h, ww1_ref[HID:2 * HID, :],
                          preferred_element_type=jnp.float32)


def _tgrid_body(at_ref, bt_ref, af_ref, bf_ref, bt1_ref, wt2_ref, bt2_ref,
                out_ref, outT_ref):
    b1 = bt1_ref[...][None, :, :]
    bf3 = bf_ref[...].reshape(S, B, HID)
    x = jnp.tanh(bf3 + at_ref[...][None, :, :] + b1)
    t = jnp.dot(x.reshape(S * B, HID), wt2_ref[...],
                preferred_element_type=jnp.float32) + bt2_ref[0, 0]
    out_ref[...] = t.reshape(1, S, B)
    af3 = af_ref[...].reshape(S, B, HID)
    xT = jnp.tanh(af3 + bt_ref[...][None, :, :] + b1)
    tT = jnp.dot(xT.reshape(S * B, HID), wt2_ref[...],
                 preferred_element_type=jnp.float32) + bt2_ref[0, 0]
    outT_ref[...] = tT.reshape(1, S, B)


def _hidpre_body(ohi_ref, ohj_ref, aw_ref, bw_ref, bw1_ref, out_ref):
    ps = (jnp.dot(ohi_ref[...], aw_ref[...],
                  preferred_element_type=jnp.float32) +
          jnp.dot(ohj_ref[...], bw_ref[...],
                  preferred_element_type=jnp.float32))
    out_ref[...] = jnp.tanh(ps + bw1_ref[...]).astype(jnp.bfloat16)


def _word_body(hid_ref, w2_ref, b2_ref, tgt_ref, se_ref, tl_ref):
    v = pl.program_id(1)

    @pl.when(v == 0)
    def _():
        se_ref[...] = jnp.zeros_like(se_ref)
        tl_ref[...] = jnp.zeros_like(tl_ref)

    logits = jnp.dot(hid_ref[...], w2_ref[...],
                     preferred_element_type=jnp.float32) + b2_ref[0]
    se_ref[0] += jnp.sum(jnp.exp(logits), axis=1, keepdims=True)
    cols = jax.lax.broadcasted_iota(jnp.int32, (MBLK, VBLK), 1) + v * VBLK
    tmask = cols == tgt_ref[0]
    tl_ref[0] += jnp.sum(jnp.where(tmask, logits, 0.0), axis=1, keepdims=True)


def _scatter_body(se_ref, tl_ref, sct_ref, wp_ref, t01_ref):
    wp_all = tl_ref[...] - jnp.log(se_ref[...])
    wp_ref[...] = jnp.dot(sct_ref[...], wp_all,
                          preferred_element_type=jnp.float32)
    t01_ref[...] = wp_all[INIT_P:INIT_P + 1, :]


def _dp_body(tg_ref, tgT_ref, wpT_ref, t01_ref, out_ref):
    wpT = wpT_ref[...]
    tg2 = tg_ref[...]
    tgT2 = tgT_ref[...]
    shT = -jnp.log(1.0 + jnp.exp(tgT2))
    rrN = -jnp.log(1.0 + jnp.exp(-tg2))

    krow = jax.lax.broadcasted_iota(jnp.int32, (S, S * B), 0)
    icol = jax.lax.broadcasted_iota(jnp.int32, (S, S * B), 1) // B
    validT = (krow > icol) & (krow <= SL - 1)
    wplT = jnp.where(validT, shT + wpT, NEG)

    t01c = t01_ref[...]
    negf = jnp.full((S, S * B), NEG)
    tk = jnp.where((krow - icol == 1) & (icol >= 1), 0.0, negf)
    tk = jnp.where((krow == 1) & (icol == 0), t01c, tk)
    tn = jnp.where((icol - krow == 1) & (krow >= 1), 0.0, negf)
    tn = jnp.where((krow == 0) & (icol == 1), t01c, tn)

    def body(g, carry):
        tk, tn = carry
        nts = pltpu.roll(tn + rrN, -B * g, 1)
        masked = jnp.where((krow > icol) & (krow < icol + g),
                           tk + wplT + nts, NEG)
        m = jnp.max(masked, axis=0, keepdims=True)
        s = jnp.sum(jnp.exp(masked - m), axis=0, keepdims=True)
        new = m + jnp.log(s)
        tk = jnp.where(krow - icol == g, new, tk)
        tn = jnp.where(icol - krow == g, pltpu.roll(new, B * g, 1), tn)
        return tk, tn

    _, tn = jax.lax.fori_loop(2, SL + 1, body, (tk, tn))
    out_ref[...] = tn[0:1, SL * B:S * B] + rrN[0:1, SL * B:S * B]


def kernel(sentence, emb_table, W_e, b_e, W_t1, b_t1, W_t2, b_t2,
           W_w1, b_w1, W_w2, b_w2):
    f32 = jnp.float32
    sent_flat = sentence.reshape(S * B, 1)

    e = pl.pallas_call(
        _embed_body,
        grid=(5,),
        in_specs=[
            pl.BlockSpec((S * B, 1), lambda v: (0, 0)),
            pl.BlockSpec((1000, EMB), lambda v: (v, 0)),
        ],
        out_specs=pl.BlockSpec((S * B, EMB), lambda v: (0, 0)),
        out_shape=jax.ShapeDtypeStruct((S * B, EMB), f32),
        compiler_params=pltpu.CompilerParams(
            dimension_semantics=("arbitrary",)),
    )(sent_flat, emb_table)

    at, bt, aw, bw = pl.pallas_call(
        _prep_body,
        out_shape=[jax.ShapeDtypeStruct((S * B, HID), f32)] * 4,
    )(e, W_e, b_e.reshape(1, HID), W_t1, W_w1)

    tgrid, tgridT = pl.pallas_call(
        _tgrid_body,
        grid=(S,),
        in_specs=[
            pl.BlockSpec((B, HID), lambda i: (i, 0)),
            pl.BlockSpec((B, HID), lambda i: (i, 0)),
            pl.BlockSpec((S * B, HID), lambda i: (0, 0)),
            pl.BlockSpec((S * B, HID), lambda i: (0, 0)),
            pl.BlockSpec((1, HID), lambda i: (0, 0)),
            pl.BlockSpec((HID, 1), lambda i: (0, 0)),
            pl.BlockSpec((1, 1), lambda i: (0, 0)),
        ],
        out_specs=[
            pl.BlockSpec((1, S, B), lambda i: (i, 0, 0)),
            pl.BlockSpec((1, S, B), lambda i: (i, 0, 0)),
        ],
        out_shape=[jax.ShapeDtypeStruct((S, S, B), f32)] * 2,
    )(at, bt, at, bt, b_t1.reshape(1, HID), W_t2, b_t2.reshape(1, 1))

    hidpre4 = pl.pallas_call(
        _hidpre_body,
        grid=(4,),
        in_specs=[
            pl.BlockSpec((NPAIR // 4, S), lambda p: (p, 0)),
            pl.BlockSpec((NPAIR // 4, S), lambda p: (p, 0)),
            pl.BlockSpec((S, B * HID), lambda p: (0, 0)),
            pl.BlockSpec((S, B * HID), lambda p: (0, 0)),
            pl.BlockSpec((1, B * HID), lambda p: (0, 0)),
        ],
        out_specs=pl.BlockSpec((NPAIR // 4, B * HID), lambda p: (p, 0)),
        out_shape=jax.ShapeDtypeStruct((NPAIR, B * HID), jnp.bfloat16),
    )(jnp.asarray(_OHI), jnp.asarray(_OHJ),
      aw.reshape(S, B * HID), bw.reshape(S, B * HID),
      jnp.tile(b_w1, B).reshape(1, B * HID))

    hidpre = hidpre4.reshape(MROWS, HID)
    w2p = jnp.pad(W_w2.astype(jnp.bfloat16), ((0, 0), (0, VPAD - VOCAB)))
    b2p = jnp.concatenate([b_w2, jnp.full((VPAD - VOCAB,), -1e9, f32)])
    b2p3 = b2p.reshape(VPAD // VBLK, 1, VBLK)
    tgt = sentence[jnp.asarray(_tgt_rows)]
    tgt3 = tgt.reshape(MROWS // MBLK, MBLK, 1)

    sumexp, tgtl = pl.pallas_call(
        _word_body,
        grid=(MROWS // MBLK, VPAD // VBLK),
        in_specs=[
            pl.BlockSpec((MBLK, HID), lambda m, v: (m, 0)),
            pl.BlockSpec((HID, VBLK), lambda m, v: (0, v)),
            pl.BlockSpec((1, 1, VBLK), lambda m, v: (v, 0, 0)),
            pl.BlockSpec((1, MBLK, 1), lambda m, v: (m, 0, 0)),
        ],
        out_specs=[
            pl.BlockSpec((1, MBLK, 1), lambda m, v: (m, 0, 0)),
            pl.BlockSpec((1, MBLK, 1), lambda m, v: (m, 0, 0)),
        ],
        out_shape=[jax.ShapeDtypeStruct((MROWS // MBLK, MBLK, 1), f32)] * 2,
        compiler_params=pltpu.CompilerParams(
            dimension_semantics=("parallel", "arbitrary")),
    )(hidpre, w2p, b2p3, tgt3)

    wpg, t01 = pl.pallas_call(
        _scatter_body,
        out_shape=[jax.ShapeDtypeStruct((S * S, B), f32),
                   jax.ShapeDtypeStruct((1, B), f32)],
    )(sumexp.reshape(NPAIR, B), tgtl.reshape(NPAIR, B), jnp.asarray(_SCT))

    out = pl.pallas_call(
        _dp_body,
        out_shape=jax.ShapeDtypeStruct((1, B), f32),
    )(tgrid.reshape(S, S * B), tgridT.reshape(S, S * B),
      wpg.reshape(S, S * B), jnp.tile(t01, (1, S)))
    return out.reshape(B)


# speedup vs baseline: 1.2895x; 1.1022x over previous
import functools

import jax
import jax.numpy as jnp
import numpy as np
from jax.experimental import pallas as pl
from jax.experimental.pallas import tpu as pltpu

S = 32
B = 8
SL = S - 1
HID = 512
EMB = 256
VOCAB = 5000
VPAD = 5120
VBLK = 512
NPAIR = 480
NREAL = 465
INIT_P = 465
MROWS = NPAIR * B
MBLK = 960
NEG = -1e30

_pi = np.array([i for i in range(SL) for j in range(i + 1, S) if j < SL],
               dtype=np.int32)
_pj = np.array([j for i in range(SL) for j in range(i + 1, S) if j < SL],
               dtype=np.int32)
_ii = np.concatenate([_pi, np.zeros(NPAIR - NREAL, np.int32)])
_jj = np.concatenate([_pj, np.zeros(NPAIR - NREAL, np.int32)])
_tgt_rows = np.concatenate([_pj + 1, np.ones(NPAIR - NREAL, np.int32)])

_OHI = np.zeros((NPAIR, S), np.float32)
_OHI[np.arange(NPAIR), _ii] = 1.0
_OHJ = np.zeros((NPAIR, S), np.float32)
_OHJ[np.arange(NPAIR), _jj] = 1.0
_SCT = np.zeros((S * S, NPAIR), np.float32)
_SCT[_pj * S + _pi, np.arange(NREAL)] = 1.0


def _embed_body(sent_ref, emb_ref, out_ref):
    v = pl.program_id(0)

    @pl.when(v == 0)
    def _():
        out_ref[...] = jnp.zeros_like(out_ref)

    cols = jax.lax.broadcasted_iota(jnp.int32, (S * B, 1000), 1) + v * 1000
    onehot = (cols == sent_ref[...]).astype(jnp.float32)
    out_ref[...] += jnp.dot(onehot, emb_ref[...],
                            preferred_element_type=jnp.float32)


def _prep_body(e_ref, we_ref, be_ref, wt1_ref, ww1_ref, bt1_ref, wt2_ref,
               bt2_ref, ohi_ref, ohj_ref, bw1_ref,
               tg_ref, tgT_ref, hp_ref, at_s, bt_s):
    h = jnp.tanh(jnp.dot(e_ref[...], we_ref[...],
                         preferred_element_type=jnp.float32) + be_ref[...])
    at_s[...] = jnp.dot(h, wt1_ref[0:HID, :],
                        preferred_element_type=jnp.float32)
    bt_s[...] = jnp.dot(h, wt1_ref[HID:2 * HID, :],
                        preferred_element_type=jnp.float32)
    aw = jnp.dot(h, ww1_ref[0:HID, :], preferred_element_type=jnp.float32)
    bw = jnp.dot(h, ww1_ref[HID:2 * HID, :],
                 preferred_element_type=jnp.float32)

    ps = (jnp.dot(ohi_ref[...], aw.reshape(S, B * HID),
                  preferred_element_type=jnp.float32) +
          jnp.dot(ohj_ref[...], bw.reshape(S, B * HID),
                  preferred_element_type=jnp.float32))
    hp_ref[...] = jnp.tanh(ps + bw1_ref[...]).astype(jnp.bfloat16)

    b1 = bt1_ref[...][None, :, :]
    wt2 = wt2_ref[...]
    bt2 = bt2_ref[0, 0]

    def body(i, _):
        a = at_s[pl.ds(i * B, B)]
        b = bt_s[pl.ds(i * B, B)]
        bf3 = bt_s[...].reshape(S, B, HID)
        x = jnp.tanh(bf3 + a[None, :, :] + b1)
        t = jnp.dot(x.reshape(S * B, HID), wt2,
                    preferred_element_type=jnp.float32) + bt2
        tg_ref[pl.ds(i, 1)] = t.reshape(1, S, B)
        af3 = at_s[...].reshape(S, B, HID)
        xT = jnp.tanh(af3 + b[None, :, :] + b1)
        tT = jnp.dot(xT.reshape(S * B, HID), wt2,
                     preferred_element_type=jnp.float32) + bt2
        tgT_ref[pl.ds(i, 1)] = tT.reshape(1, S, B)
        return 0

    jax.lax.fori_loop(0, S, body, 0)


def _word_body(hid_ref, w2_ref, b2_ref, tgt_ref, se_ref, tl_ref):
    v = pl.program_id(1)

    @pl.when(v == 0)
    def _():
        se_ref[...] = jnp.zeros_like(se_ref)
        tl_ref[...] = jnp.zeros_like(tl_ref)

    logits = jnp.dot(hid_ref[...], w2_ref[...],
                     preferred_element_type=jnp.float32) + b2_ref[0]
    se_ref[0] += jnp.sum(jnp.exp(logits), axis=1, keepdims=True)
    cols = jax.lax.broadcasted_iota(jnp.int32, (MBLK, VBLK), 1) + v * VBLK
    tmask = cols == tgt_ref[0]
    tl_ref[0] += jnp.sum(jnp.where(tmask, logits, 0.0), axis=1, keepdims=True)


def _scatter_body(se_ref, tl_ref, sct_ref, wp_ref, t01_ref):
    wp_all = tl_ref[...] - jnp.log(se_ref[...])
    wp_ref[...] = jnp.dot(sct_ref[...], wp_all,
                          preferred_element_type=jnp.float32)
    t01_ref[...] = wp_all[INIT_P:INIT_P + 1, :]


def _dp_body(tg_ref, tgT_ref, wpT_ref, t01_ref, out_ref):
    wpT = wpT_ref[...]
    tg2 = tg_ref[...]
    tgT2 = tgT_ref[...]
    shT = -jnp.log(1.0 + jnp.exp(tgT2))
    rrN = -jnp.log(1.0 + jnp.exp(-tg2))

    krow = jax.lax.broadcasted_iota(jnp.int32, (S, S * B), 0)
    icol = jax.lax.broadcasted_iota(jnp.int32, (S, S * B), 1) // B
    validT = (krow > icol) & (krow <= SL - 1)
    wplT = jnp.where(validT, shT + wpT, NEG)

    t01c = t01_ref[...]
    negf = jnp.full((S, S * B), NEG)
    tk = jnp.where((krow - icol == 1) & (icol >= 1), 0.0, negf)
    tk = jnp.where((krow == 1) & (icol == 0), t01c, tk)
    tn = jnp.where((icol - krow == 1) & (krow >= 1), 0.0, negf)
    tn = jnp.where((krow == 0) & (icol == 1), t01c, tn)

    def body(g, carry):
        tk, tn = carry
        nts = pltpu.roll(tn + rrN, -B * g, 1)
        masked = jnp.where((krow > icol) & (krow < icol + g),
                           tk + wplT + nts, NEG)
        m = jnp.max(masked, axis=0, keepdims=True)
        s = jnp.sum(jnp.exp(masked - m), axis=0, keepdims=True)
        new = m + jnp.log(s)
        tk = jnp.where(krow - icol == g, new, tk)
        tn = jnp.where(icol - krow == g, pltpu.roll(new, B * g, 1), tn)
        return tk, tn

    _, tn = jax.lax.fori_loop(2, SL + 1, body, (tk, tn))
    out_ref[...] = tn[0:1, SL * B:S * B] + rrN[0:1, SL * B:S * B]


def kernel(sentence, emb_table, W_e, b_e, W_t1, b_t1, W_t2, b_t2,
           W_w1, b_w1, W_w2, b_w2):
    f32 = jnp.float32
    sent_flat = sentence.reshape(S * B, 1)

    e = pl.pallas_call(
        _embed_body,
        grid=(5,),
        in_specs=[
            pl.BlockSpec((S * B, 1), lambda v: (0, 0)),
            pl.BlockSpec((1000, EMB), lambda v: (v, 0)),
        ],
        out_specs=pl.BlockSpec((S * B, EMB), lambda v: (0, 0)),
        out_shape=jax.ShapeDtypeStruct((S * B, EMB), f32),
        compiler_params=pltpu.CompilerParams(
            dimension_semantics=("arbitrary",)),
    )(sent_flat, emb_table)

    tgrid, tgridT, hidpre4 = pl.pallas_call(
        _prep_body,
        out_shape=[jax.ShapeDtypeStruct((S, S, B), f32),
                   jax.ShapeDtypeStruct((S, S, B), f32),
                   jax.ShapeDtypeStruct((NPAIR, B * HID), jnp.bfloat16)],
        scratch_shapes=[pltpu.VMEM((S * B, HID), f32)] * 2,
    )(e, W_e, b_e.reshape(1, HID), W_t1, W_w1, b_t1.reshape(1, HID),
      W_t2, b_t2.reshape(1, 1), jnp.asarray(_OHI), jnp.asarray(_OHJ),
      jnp.tile(b_w1, B).reshape(1, B * HID))

    hidpre = hidpre4.reshape(MROWS, HID)
    w2p = jnp.pad(W_w2.astype(jnp.bfloat16), ((0, 0), (0, VPAD - VOCAB)))
    b2p = jnp.concatenate([b_w2, jnp.full((VPAD - VOCAB,), -1e9, f32)])
    b2p3 = b2p.reshape(VPAD // VBLK, 1, VBLK)
    tgt = sentence[jnp.asarray(_tgt_rows)]
    tgt3 = tgt.reshape(MROWS // MBLK, MBLK, 1)

    sumexp, tgtl = pl.pallas_call(
        _word_body,
        grid=(MROWS // MBLK, VPAD // VBLK),
        in_specs=[
            pl.BlockSpec((MBLK, HID), lambda m, v: (m, 0)),
            pl.BlockSpec((HID, VBLK), lambda m, v: (0, v)),
            pl.BlockSpec((1, 1, VBLK), lambda m, v: (v, 0, 0)),
            pl.BlockSpec((1, MBLK, 1), lambda m, v: (m, 0, 0)),
        ],
        out_specs=[
            pl.BlockSpec((1, MBLK, 1), lambda m, v: (m, 0, 0)),
            pl.BlockSpec((1, MBLK, 1), lambda m, v: (m, 0, 0)),
        ],
        out_shape=[jax.ShapeDtypeStruct((MROWS // MBLK, MBLK, 1), f32)] * 2,
        compiler_params=pltpu.CompilerParams(
            dimension_semantics=("parallel", "arbitrary")),
    )(hidpre, w2p, b2p3, tgt3)

    wpg, t01 = pl.pallas_call(
        _scatter_body,
        out_shape=[jax.ShapeDtypeStruct((S * S, B), f32),
                   jax.ShapeDtypeStruct((1, B), f32)],
    )(sumexp.reshape(NPAIR, B), tgtl.reshape(NPAIR, B), jnp.asarray(_SCT))

    out = pl.pallas_call(
        _dp_body,
        out_shape=jax.ShapeDtypeStruct((1, B), f32),
    )(tgrid.reshape(S, S * B), tgridT.reshape(S, S * B),
      wpg.reshape(S, S * B), jnp.tile(t01, (1, S)))
    return out.reshape(B)


# speedup vs baseline: 1.3997x; 1.0854x over previous
import functools

import jax
import jax.numpy as jnp
import numpy as np
from jax.experimental import pallas as pl
from jax.experimental.pallas import tpu as pltpu

S = 32
B = 8
SL = S - 1
HID = 512
EMB = 256
VOCAB = 5000
VPAD = 5120
VBLK = 512
NPAIR = 480
NREAL = 465
INIT_P = 465
MROWS = NPAIR * B
MBLK = 960
NEG = -1e30

_pi = np.array([i for i in range(SL) for j in range(i + 1, S) if j < SL],
               dtype=np.int32)
_pj = np.array([j for i in range(SL) for j in range(i + 1, S) if j < SL],
               dtype=np.int32)
_ii = np.concatenate([_pi, np.zeros(NPAIR - NREAL, np.int32)])
_jj = np.concatenate([_pj, np.zeros(NPAIR - NREAL, np.int32)])
_tgt_rows = np.concatenate([_pj + 1, np.ones(NPAIR - NREAL, np.int32)])

_OHI = np.zeros((NPAIR, S), np.float32)
_OHI[np.arange(NPAIR), _ii] = 1.0
_OHJ = np.zeros((NPAIR, S), np.float32)
_OHJ[np.arange(NPAIR), _jj] = 1.0
_SCK = np.zeros((S, NPAIR), np.float32)
_SCK[_pj, np.arange(NREAL)] = 1.0
_OHIX = np.zeros((NPAIR, S * B), np.float32)
_OHIX[np.arange(NREAL)[:, None],
      (_pi[:, None] * B + np.arange(B)[None, :])] = 1.0
_TILE8 = np.tile(np.eye(B, dtype=np.float32), (1, S))


def _embed_body(sent_ref, emb_ref, out_ref):
    v = pl.program_id(0)

    @pl.when(v == 0)
    def _():
        out_ref[...] = jnp.zeros_like(out_ref)

    cols = jax.lax.broadcasted_iota(jnp.int32, (S * B, 1000), 1) + v * 1000
    onehot = (cols == sent_ref[...]).astype(jnp.float32)
    out_ref[...] += jnp.dot(onehot, emb_ref[...],
                            preferred_element_type=jnp.float32)


def _prep_body(e_ref, we_ref, be_ref, wt1_ref, ww1_ref, bt1_ref, wt2_ref,
               bt2_ref, ohi_ref, ohj_ref, bw1_ref,
               tg_ref, tgT_ref, hp_ref):
    h = jnp.tanh(jnp.dot(e_ref[...], we_ref[...],
                         preferred_element_type=jnp.float32) + be_ref[...])
    at = jnp.dot(h, wt1_ref[0:HID, :], preferred_element_type=jnp.float32)
    bt = jnp.dot(h, wt1_ref[HID:2 * HID, :],
                 preferred_element_type=jnp.float32)
    aw = jnp.dot(h, ww1_ref[0:HID, :], preferred_element_type=jnp.float32)
    bw = jnp.dot(h, ww1_ref[HID:2 * HID, :],
                 preferred_element_type=jnp.float32)

    ps = (jnp.dot(ohi_ref[...], aw.reshape(S, B * HID),
                  preferred_element_type=jnp.float32) +
          jnp.dot(ohj_ref[...], bw.reshape(S, B * HID),
                  preferred_element_type=jnp.float32))
    hp_ref[...] = jnp.tanh(ps + bw1_ref[...]).astype(jnp.bfloat16)

    b1 = bt1_ref[...].reshape(1, 1, 1, HID)
    wt2 = wt2_ref[...].astype(jnp.bfloat16)
    bt2 = bt2_ref[0, 0]
    a4i = at.reshape(S, 1, B, HID)
    b4j = bt.reshape(1, S, B, HID)
    x = jnp.tanh(jnp.broadcast_to(a4i, (S, S, B, HID)) +
                 jnp.broadcast_to(b4j, (S, S, B, HID)) +
                 b1).astype(jnp.bfloat16).reshape(S * S * B, HID)
    tg_ref[...] = jnp.dot(x, wt2, preferred_element_type=jnp.float32) + bt2
    a4j = at.reshape(1, S, B, HID)
    b4i = bt.reshape(S, 1, B, HID)
    xT = jnp.tanh(jnp.broadcast_to(b4i, (S, S, B, HID)) +
                  jnp.broadcast_to(a4j, (S, S, B, HID)) +
                  b1).astype(jnp.bfloat16).reshape(S * S * B, HID)
    tgT_ref[...] = jnp.dot(xT, wt2, preferred_element_type=jnp.float32) + bt2


def _word_body(hid_ref, w2_ref, b2_ref, tgt_ref, se_ref, tl_ref):
    v = pl.program_id(1)

    @pl.when(v == 0)
    def _():
        se_ref[...] = jnp.zeros_like(se_ref)
        tl_ref[...] = jnp.zeros_like(tl_ref)

    logits = jnp.dot(hid_ref[...], w2_ref[...],
                     preferred_element_type=jnp.float32) + b2_ref[0]
    se_ref[0] += jnp.sum(jnp.exp(logits), axis=1, keepdims=True)
    cols = jax.lax.broadcasted_iota(jnp.int32, (MBLK, VBLK), 1) + v * VBLK
    tmask = cols == tgt_ref[0]
    tl_ref[0] += jnp.sum(jnp.where(tmask, logits, 0.0), axis=1, keepdims=True)


def _dp_body(tg_ref, tgT_ref, se_ref, tl_ref, sck_ref, ohix_ref, tile8_ref,
             out_ref):
    wp_all = tl_ref[...] - jnp.log(se_ref[...])
    wp_bc = jnp.dot(wp_all, tile8_ref[...],
                    preferred_element_type=jnp.float32)
    wpT = jnp.dot(sck_ref[...], wp_bc * ohix_ref[...],
                  preferred_element_type=jnp.float32)
    t01c = jnp.dot(wp_all[INIT_P:INIT_P + 1, :], tile8_ref[...],
                   preferred_element_type=jnp.float32)
    tg2 = tg_ref[...]
    tgT2 = tgT_ref[...]
    shT = -jnp.log(1.0 + jnp.exp(tgT2))
    rrN = -jnp.log(1.0 + jnp.exp(-tg2))

    krow = jax.lax.broadcasted_iota(jnp.int32, (S, S * B), 0)
    icol = jax.lax.broadcasted_iota(jnp.int32, (S, S * B), 1) // B
    validT = (krow > icol) & (krow <= SL - 1)
    wplT = jnp.where(validT, shT + wpT, NEG)

    negf = jnp.full((S, S * B), NEG)
    tk = jnp.where((krow - icol == 1) & (icol >= 1), 0.0, negf)
    tk = jnp.where((krow == 1) & (icol == 0), t01c, tk)
    tn = jnp.where((icol - krow == 1) & (krow >= 1), 0.0, negf)
    tn = jnp.where((krow == 0) & (icol == 1), t01c, tn)

    def body(g, carry):
        tk, tn = carry
        nts = pltpu.roll(tn + rrN, -B * g, 1)
        masked = jnp.where((krow > icol) & (krow < icol + g),
                           tk + wplT + nts, NEG)
        m = jnp.max(masked, axis=0, keepdims=True)
        s = jnp.sum(jnp.exp(masked - m), axis=0, keepdims=True)
        new = m + jnp.log(s)
        tk = jnp.where(krow - icol == g, new, tk)
        tn = jnp.where(icol - krow == g, pltpu.roll(new, B * g, 1), tn)
        return tk, tn

    _, tn = jax.lax.fori_loop(2, SL + 1, body, (tk, tn))
    out_ref[...] = tn[0:1, SL * B:S * B] + rrN[0:1, SL * B:S * B]


def kernel(sentence, emb_table, W_e, b_e, W_t1, b_t1, W_t2, b_t2,
           W_w1, b_w1, W_w2, b_w2):
    f32 = jnp.float32
    sent_flat = sentence.reshape(S * B, 1)

    e = pl.pallas_call(
        _embed_body,
        grid=(5,),
        in_specs=[
            pl.BlockSpec((S * B, 1), lambda v: (0, 0)),
            pl.BlockSpec((1000, EMB), lambda v: (v, 0)),
        ],
        out_specs=pl.BlockSpec((S * B, EMB), lambda v: (0, 0)),
        out_shape=jax.ShapeDtypeStruct((S * B, EMB), f32),
        compiler_params=pltpu.CompilerParams(
            dimension_semantics=("arbitrary",)),
    )(sent_flat, emb_table)

    tgrid, tgridT, hidpre4 = pl.pallas_call(
        _prep_body,
        out_shape=[jax.ShapeDtypeStruct((S * S * B, 1), f32),
                   jax.ShapeDtypeStruct((S * S * B, 1), f32),
                   jax.ShapeDtypeStruct((NPAIR, B * HID), jnp.bfloat16)],
    )(e, W_e, b_e.reshape(1, HID), W_t1, W_w1, b_t1.reshape(1, HID),
      W_t2, b_t2.reshape(1, 1), jnp.asarray(_OHI), jnp.asarray(_OHJ),
      jnp.tile(b_w1, B).reshape(1, B * HID))

    hidpre = hidpre4.reshape(MROWS, HID)
    w2p = jnp.pad(W_w2.astype(jnp.bfloat16), ((0, 0), (0, VPAD - VOCAB)))
    b2p = jnp.concatenate([b_w2, jnp.full((VPAD - VOCAB,), -1e9, f32)])
    b2p3 = b2p.reshape(VPAD // VBLK, 1, VBLK)
    tgt = sentence[jnp.asarray(_tgt_rows)]
    tgt3 = tgt.reshape(MROWS // MBLK, MBLK, 1)

    sumexp, tgtl = pl.pallas_call(
        _word_body,
        grid=(MROWS // MBLK, VPAD // VBLK),
        in_specs=[
            pl.BlockSpec((MBLK, HID), lambda m, v: (m, 0)),
            pl.BlockSpec((HID, VBLK), lambda m, v: (0, v)),
            pl.BlockSpec((1, 1, VBLK), lambda m, v: (v, 0, 0)),
            pl.BlockSpec((1, MBLK, 1), lambda m, v: (m, 0, 0)),
        ],
        out_specs=[
            pl.BlockSpec((1, MBLK, 1), lambda m, v: (m, 0, 0)),
            pl.BlockSpec((1, MBLK, 1), lambda m, v: (m, 0, 0)),
        ],
        out_shape=[jax.ShapeDtypeStruct((MROWS // MBLK, MBLK, 1), f32)] * 2,
        compiler_params=pltpu.CompilerParams(
            dimension_semantics=("parallel", "arbitrary")),
    )(hidpre, w2p, b2p3, tgt3)

    out = pl.pallas_call(
        _dp_body,
        out_shape=jax.ShapeDtypeStruct((1, B), f32),
    )(tgrid.reshape(S, S * B), tgridT.reshape(S, S * B),
      sumexp.reshape(NPAIR, B), tgtl.reshape(NPAIR, B),
      jnp.asarray(_SCK), jnp.asarray(_OHIX), jnp.asarray(_TILE8))
    return out.reshape(B)
